# Initial kernel scaffold; baseline (speedup 1.0000x reference)
import jax
import jax.numpy as jnp
from jax.experimental import pallas as pl


def kernel(x, enc_w1, enc_b1, enc_w2, enc_b2, enc_w3, enc_b3, dec_w1, dec_b1, dec_w2, dec_b2, dec_w3, dec_b3, codebooks):
    raise NotImplementedError("write your pallas kernel here")


# speedup vs baseline: 2.0307x; 2.0307x over previous
import jax
import jax.numpy as jnp
from jax.experimental import pallas as pl
from jax.experimental.pallas import tpu as pltpu

B = 4096
D_IN = 768
H = 512
D_LAT = 64
K = 8192
L = 3
BETA = 0.25

BM = 256


def _rqvae_kernel(x_ref, ew1, eb1, ew2, eb2, ew3, eb3,
                  dw1, db1, dw2, db2, dw3, db3, cb_ref,
                  xhat_ref, codes_ref, loss_ref,
                  cb3_ref, css_ref):
    i = pl.program_id(0)

    @pl.when(i == 0)
    def _():
        for l in range(L):
            cb = cb_ref[l]
            hi = cb.astype(jnp.bfloat16)
            r1 = cb - hi.astype(jnp.float32)
            mid = r1.astype(jnp.bfloat16)
            r2 = r1 - mid.astype(jnp.float32)
            lo = r2.astype(jnp.bfloat16)
            cb3_ref[l] = jnp.concatenate([hi, mid, lo], axis=1)
            css_ref[l] = jnp.sum(cb * cb, axis=1)[None, :]

    x = x_ref[...]

    h = jnp.maximum(jnp.dot(x, ew1[...], preferred_element_type=jnp.float32) + eb1[...], 0.0)
    h = jnp.maximum(jnp.dot(h, ew2[...], preferred_element_type=jnp.float32) + eb2[...], 0.0)
    z = jnp.dot(h, ew3[...], preferred_element_type=jnp.float32) + eb3[...]

    residual = z
    qtot = jnp.zeros_like(z)
    closs = jnp.float32(0.0)
    codes_rows = []
    iota = jax.lax.broadcasted_iota(jnp.int32, (BM, K), 1)
    iota16 = jax.lax.broadcasted_iota(jnp.int16, (BM, K), 1)
    for l in range(L):
        cb = cb_ref[l]
        rss = jnp.sum(residual * residual, axis=1, keepdims=True)
        prod2 = jax.lax.dot_general(residual * -2.0, cb, (((1,), (1,)), ((), ())),
                                    preferred_element_type=jnp.float32)
        d = (rss + prod2) + css_ref[l]
        dmin = jnp.min(d, axis=1, keepdims=True)
        codes_l = jnp.min(jnp.where(d == dmin, iota, K), axis=1)

        oh = jnp.where(iota16 == codes_l.astype(jnp.int16)[:, None],
                       jnp.bfloat16(1), jnp.bfloat16(0))
        e3 = jnp.dot(oh, cb3_ref[l], preferred_element_type=jnp.float32)
        e_k = (e3[:, :D_LAT] + e3[:, D_LAT:2 * D_LAT]) + e3[:, 2 * D_LAT:]

        closs = closs + jnp.sum((residual - e_k) ** 2)
        qtot = qtot + e_k
        residual = residual - e_k
        codes_rows.append(codes_l)

    codes_ref[...] = jnp.stack(codes_rows, axis=0)

    h = jnp.maximum(jnp.dot(qtot, dw1[...], preferred_element_type=jnp.float32) + db1[...], 0.0)
    h = jnp.maximum(jnp.dot(h, dw2[...], preferred_element_type=jnp.float32) + db2[...], 0.0)
    xh = jnp.dot(h, dw3[...], preferred_element_type=jnp.float32) + db3[...]
    xhat_ref[...] = xh

    part = (jnp.sum((x - xh) ** 2) / (B * D_IN)
            + BETA * closs / (B * D_LAT)).reshape(1, 1)

    @pl.when(i == 0)
    def _():
        loss_ref[...] = part

    @pl.when(i != 0)
    def _():
        loss_ref[...] = loss_ref[...] + part


def kernel(x, enc_w1, enc_b1, enc_w2, enc_b2, enc_w3, enc_b3,
           dec_w1, dec_b1, dec_w2, dec_b2, dec_w3, dec_b3, codebooks):
    grid = (B // BM,)
    full = lambda shape: pl.BlockSpec(shape, lambda i: tuple(0 for _ in shape))
    x_hat, codes_t, loss = pl.pallas_call(
        _rqvae_kernel,
        grid=grid,
        in_specs=[
            pl.BlockSpec((BM, D_IN), lambda i: (i, 0)),
            full((D_IN, H)), full((1, H)),
            full((H, H)), full((1, H)),
            full((H, D_LAT)), full((1, D_LAT)),
            full((D_LAT, H)), full((1, H)),
            full((H, H)), full((1, H)),
            full((H, D_IN)), full((1, D_IN)),
            full((L, K, D_LAT)),
        ],
        out_specs=[
            pl.BlockSpec((BM, D_IN), lambda i: (i, 0)),
            pl.BlockSpec((L, BM), lambda i: (0, i)),
            pl.BlockSpec((1, 1), lambda i: (0, 0)),
        ],
        out_shape=[
            jax.ShapeDtypeStruct((B, D_IN), jnp.float32),
            jax.ShapeDtypeStruct((L, B), jnp.int32),
            jax.ShapeDtypeStruct((1, 1), jnp.float32),
        ],
        scratch_shapes=[
            pltpu.VMEM((L, K, 3 * D_LAT), jnp.bfloat16),
            pltpu.VMEM((L, 1, K), jnp.float32),
        ],
        compiler_params=pltpu.CompilerParams(
            dimension_semantics=("arbitrary",),
        ),
    )(x, enc_w1, enc_b1.reshape(1, H), enc_w2, enc_b2.reshape(1, H),
      enc_w3, enc_b3.reshape(1, D_LAT), dec_w1, dec_b1.reshape(1, H),
      dec_w2, dec_b2.reshape(1, H), dec_w3, dec_b3.reshape(1, D_IN),
      codebooks)
    return x_hat, codes_t.T, loss[0, 0]


# speedup vs baseline: 2.2617x; 1.1137x over previous
import jax
import jax.numpy as jnp
from jax.experimental import pallas as pl
from jax.experimental.pallas import tpu as pltpu

B = 4096
D_IN = 768
H = 512
D_LAT = 64
K = 8192
L = 3
BETA = 0.25

BM = 256
AUGW = 256


def _rqvae_kernel(x_ref, ew1, eb1, ew2, eb2, ew3, eb3,
                  dw1, db1, dw2, db2, dw3, db3, cb_ref,
                  xhat_ref, codes_ref, loss_ref,
                  aug_ref, css_ref):
    i = pl.program_id(0)

    @pl.when(i == 0)
    def _():
        iota_col = jax.lax.broadcasted_iota(jnp.int32, (K, 1), 0).astype(jnp.float32)
        ihi = iota_col.astype(jnp.bfloat16)
        ilo = (iota_col - ihi.astype(jnp.float32)).astype(jnp.bfloat16)
        ones = jnp.ones((K, 1), jnp.bfloat16)
        pad = jnp.zeros((K, AUGW - 3 * D_LAT - 3), jnp.bfloat16)
        for l in range(L):
            cb = cb_ref[l]
            hi = cb.astype(jnp.bfloat16)
            r1 = cb - hi.astype(jnp.float32)
            mid = r1.astype(jnp.bfloat16)
            r2 = r1 - mid.astype(jnp.float32)
            lo = r2.astype(jnp.bfloat16)
            aug_ref[l] = jnp.concatenate([hi, mid, lo, ihi, ilo, ones, pad],
                                         axis=1)
            css_ref[l] = jnp.sum(cb * cb, axis=1)[None, :]

    x = x_ref[...]

    h = jnp.maximum(jnp.dot(x, ew1[...], preferred_element_type=jnp.float32) + eb1[...], 0.0)
    h = jnp.maximum(jnp.dot(h, ew2[...], preferred_element_type=jnp.float32) + eb2[...], 0.0)
    z = jnp.dot(h, ew3[...], preferred_element_type=jnp.float32) + eb3[...]

    residual = z
    qtot = jnp.zeros_like(z)
    closs = jnp.float32(0.0)
    codes_rows = []
    for l in range(L):
        cb = cb_ref[l]
        rss = jnp.sum(residual * residual, axis=1, keepdims=True)
        prod2 = jax.lax.dot_general(residual * -2.0, cb, (((1,), (1,)), ((), ())),
                                    preferred_element_type=jnp.float32)
        d = (rss + prod2) + css_ref[l]
        dmin = jnp.min(d, axis=1, keepdims=True)
        m = jnp.where(d == dmin, 1.0, 0.0).astype(jnp.bfloat16)
        e3 = jnp.dot(m, aug_ref[l], preferred_element_type=jnp.float32)
        cnt = e3[:, 3 * D_LAT + 2]

        def _unique(d=d, dmin=dmin, e3=e3):
            e_k = (e3[:, :D_LAT] + e3[:, D_LAT:2 * D_LAT]) + e3[:, 2 * D_LAT:3 * D_LAT]
            codes_l = (e3[:, 3 * D_LAT] + e3[:, 3 * D_LAT + 1]).astype(jnp.int32)
            return codes_l, e_k

        def _tie(d=d, dmin=dmin):
            iota = jax.lax.broadcasted_iota(jnp.int32, (BM, K), 1)
            codes_l = jnp.min(jnp.where(d == dmin, iota, K), axis=1)
            iota16 = jax.lax.broadcasted_iota(jnp.int16, (BM, K), 1)
            oh = jnp.where(iota16 == codes_l.astype(jnp.int16)[:, None],
                           jnp.bfloat16(1), jnp.bfloat16(0))
            e3b = jnp.dot(oh, aug_ref[l], preferred_element_type=jnp.float32)
            e_k = (e3b[:, :D_LAT] + e3b[:, D_LAT:2 * D_LAT]) + e3b[:, 2 * D_LAT:3 * D_LAT]
            return codes_l, e_k

        codes_l, e_k = jax.lax.cond(jnp.any(cnt > 1.5), _tie, _unique)

        closs = closs + jnp.sum((residual - e_k) ** 2)
        qtot = qtot + e_k
        residual = residual - e_k
        codes_rows.append(codes_l)

    codes_ref[...] = jnp.stack(codes_rows, axis=0)

    h = jnp.maximum(jnp.dot(qtot, dw1[...], preferred_element_type=jnp.float32) + db1[...], 0.0)
    h = jnp.maximum(jnp.dot(h, dw2[...], preferred_element_type=jnp.float32) + db2[...], 0.0)
    xh = jnp.dot(h, dw3[...], preferred_element_type=jnp.float32) + db3[...]
    xhat_ref[...] = xh

    part = (jnp.sum((x - xh) ** 2) / (B * D_IN)
            + BETA * closs / (B * D_LAT)).reshape(1, 1)

    @pl.when(i == 0)
    def _():
        loss_ref[...] = part

    @pl.when(i != 0)
    def _():
        loss_ref[...] = loss_ref[...] + part


def kernel(x, enc_w1, enc_b1, enc_w2, enc_b2, enc_w3, enc_b3,
           dec_w1, dec_b1, dec_w2, dec_b2, dec_w3, dec_b3, codebooks):
    grid = (B // BM,)
    full = lambda shape: pl.BlockSpec(shape, lambda i: tuple(0 for _ in shape))
    x_hat, codes_t, loss = pl.pallas_call(
        _rqvae_kernel,
        grid=grid,
        in_specs=[
            pl.BlockSpec((BM, D_IN), lambda i: (i, 0)),
            full((D_IN, H)), full((1, H)),
            full((H, H)), full((1, H)),
            full((H, D_LAT)), full((1, D_LAT)),
            full((D_LAT, H)), full((1, H)),
            full((H, H)), full((1, H)),
            full((H, D_IN)), full((1, D_IN)),
            full((L, K, D_LAT)),
        ],
        out_specs=[
            pl.BlockSpec((BM, D_IN), lambda i: (i, 0)),
            pl.BlockSpec((L, BM), lambda i: (0, i)),
            pl.BlockSpec((1, 1), lambda i: (0, 0)),
        ],
        out_shape=[
            jax.ShapeDtypeStruct((B, D_IN), jnp.float32),
            jax.ShapeDtypeStruct((L, B), jnp.int32),
            jax.ShapeDtypeStruct((1, 1), jnp.float32),
        ],
        scratch_shapes=[
            pltpu.VMEM((L, K, AUGW), jnp.bfloat16),
            pltpu.VMEM((L, 1, K), jnp.float32),
        ],
        compiler_params=pltpu.CompilerParams(
            dimension_semantics=("arbitrary",),
        ),
    )(x, enc_w1, enc_b1.reshape(1, H), enc_w2, enc_b2.reshape(1, H),
      enc_w3, enc_b3.reshape(1, D_LAT), dec_w1, dec_b1.reshape(1, H),
      dec_w2, dec_b2.reshape(1, H), dec_w3, dec_b3.reshape(1, D_IN),
      codebooks)
    return x_hat, codes_t.T, loss[0, 0]
